# baseline (speedup 1.0000x reference)
import functools

import jax
import jax.numpy as jnp
from jax import lax
from jax.experimental import pallas as pl
from jax.experimental.pallas import tpu as pltpu
from jax.experimental.pallas import tpu_sc as plsc

T = 2048
DIM = 1024
HID = 4096
E = 8
TOP_K = 2

BM = 256
NPAD = 6144
NB = NPAD // BM

NC = 2
NS = 16
NW = NC * NS
W32 = DIM // 2
IDXCH = 64

_G_PER_W = NPAD // NW
_C_PER_W = T // NW

_INV_SQRT2 = 0.7071067811865476


def _gate_body(x_ref, gw_ref, gb_ref, ev_ref, wv_ref):
    xf = x_ref[...]
    logits = lax.dot_general(
        xf, gw_ref[...], (((1,), (1,)), ((), ())),
        preferred_element_type=jnp.float32) + gb_ref[...]
    mx = jnp.max(logits, axis=-1, keepdims=True)
    ex = jnp.exp(logits - mx)
    p = ex / jnp.sum(ex, axis=-1, keepdims=True)
    iota8 = lax.broadcasted_iota(jnp.int32, p.shape, 1)
    v1 = jnp.max(p, axis=-1, keepdims=True)
    i1 = jnp.min(jnp.where(p >= v1, iota8, E), axis=-1, keepdims=True)
    m1 = iota8 == i1
    p2 = jnp.where(m1, -jnp.inf, p)
    v2 = jnp.max(p2, axis=-1, keepdims=True)
    i2 = jnp.min(jnp.where(p2 >= v2, iota8, E), axis=-1, keepdims=True)
    ev_ref[...] = jnp.concatenate([i1, i2], axis=1)
    wv_ref[...] = jnp.concatenate([v1, v2], axis=1)


def _gating(x, gate_w, gate_b2d):
    return pl.pallas_call(
        _gate_body,
        grid=(1,),
        in_specs=[
            pl.BlockSpec((T, DIM), lambda i: (0, 0)),
            pl.BlockSpec((E, DIM), lambda i: (0, 0)),
            pl.BlockSpec((1, E), lambda i: (0, 0)),
        ],
        out_specs=[
            pl.BlockSpec((T, TOP_K), lambda i: (0, 0)),
            pl.BlockSpec((T, TOP_K), lambda i: (0, 0)),
        ],
        out_shape=[
            jax.ShapeDtypeStruct((T, TOP_K), jnp.int32),
            jax.ShapeDtypeStruct((T, TOP_K), jnp.float32),
        ],
    )(x, gate_w, gate_b2d)


def _route_meta(ev, wv):
    fe = ev.reshape(-1)
    wf = wv.reshape(-1)
    oh = fe[:, None] == jnp.arange(E, dtype=jnp.int32)[None, :]
    csum = jnp.cumsum(oh.astype(jnp.int32), axis=0)
    counts = csum[-1]
    rank = jnp.sum(jnp.where(oh, csum - 1, 0), axis=1)
    padded = ((counts + BM - 1) // BM) * BM
    ends = jnp.cumsum(padded)
    offs = ends - padded
    dest = offs[fe] + rank
    row_token = jnp.zeros((NPAD,), jnp.int32).at[dest].set(
        jnp.arange(2 * T, dtype=jnp.int32) // TOP_K)
    row_weight = jnp.zeros((NPAD,), jnp.float32).at[dest].set(wf)
    bstart = jnp.arange(NB, dtype=jnp.int32) * BM
    block_expert = jnp.minimum(
        jnp.sum(bstart[:, None] >= ends[None, :], axis=1), E - 1
    ).astype(jnp.int32)
    nactive = (ends[-1] // BM).astype(jnp.int32)
    pe_plus = jnp.concatenate([block_expert, nactive[None]])
    pos0 = dest[0::2].astype(jnp.int32)
    pos1 = dest[1::2].astype(jnp.int32)
    return pe_plus, row_token, row_weight.reshape(NPAD, 1), pos0, pos1


_SC_BUILT = {}


def _build_sc_kernels():
    if "gather" in _SC_BUILT:
        return _SC_BUILT
    mesh = plsc.VectorSubcoreMesh(core_axis_name="c", subcore_axis_name="s")

    @functools.partial(
        pl.kernel, mesh=mesh,
        out_type=jax.ShapeDtypeStruct((NPAD, W32), jnp.int32),
        scratch_types=[
            pltpu.VMEM((_G_PER_W,), jnp.int32),
            pltpu.VMEM((_G_PER_W, W32), jnp.int32),
            pltpu.SemaphoreType.DMA,
        ],
    )
    def sc_gather(x_hbm, idx_hbm, out_hbm, idx_v, rows_v, sem):
        wid = lax.axis_index("s") * NC + lax.axis_index("c")
        base = wid * _G_PER_W
        pltpu.sync_copy(idx_hbm.at[pl.ds(base, _G_PER_W)], idx_v)
        cps = []
        for ci in range(_G_PER_W // IDXCH):
            cps.append(pltpu.async_copy(
                x_hbm.at[idx_v.at[pl.ds(ci * IDXCH, IDXCH)]],
                rows_v.at[pl.ds(ci * IDXCH, IDXCH)], sem))
        for cp in cps:
            cp.wait()
        pltpu.sync_copy(rows_v, out_hbm.at[pl.ds(base, _G_PER_W)])

    CHK = 32

    @functools.partial(
        pl.kernel, mesh=mesh,
        out_type=jax.ShapeDtypeStruct((T, DIM), jnp.float32),
        scratch_types=[
            pltpu.VMEM((CHK,), jnp.int32),
            pltpu.VMEM((CHK,), jnp.int32),
            pltpu.VMEM((CHK, DIM), jnp.float32),
            pltpu.VMEM((CHK, DIM), jnp.float32),
            pltpu.SemaphoreType.DMA,
        ],
    )
    def sc_combine(ys_hbm, pos0_hbm, pos1_hbm, out_hbm,
                   posa, posb, rows_a, rows_b, sem):
        c = lax.axis_index("c")
        s = lax.axis_index("s")
        tokbase = c * (NS * _C_PER_W) + s * _C_PER_W
        for ci in range(_C_PER_W // CHK):
            cbase = tokbase + ci * CHK
            pltpu.sync_copy(pos0_hbm.at[pl.ds(cbase, CHK)], posa)
            pltpu.sync_copy(pos1_hbm.at[pl.ds(cbase, CHK)], posb)
            cpa = pltpu.async_copy(ys_hbm.at[posa], rows_a, sem)
            cpb = pltpu.async_copy(ys_hbm.at[posb], rows_b, sem)
            cpa.wait()
            cpb.wait()

            def addrow(r, carry):
                for u in range(DIM // 16):
                    sl = pl.ds(u * 16, 16)
                    rows_a[r, sl] = rows_a[r, sl] + rows_b[r, sl]
                return carry

            lax.fori_loop(0, CHK, addrow, 0)
            pltpu.sync_copy(rows_a, out_hbm.at[pl.ds(cbase, CHK)])

    _SC_BUILT["gather"] = sc_gather
    _SC_BUILT["combine"] = sc_combine
    return _SC_BUILT


def _sc_gather(x_i32, row_token):
    return _build_sc_kernels()["gather"](x_i32, row_token)


def _sc_combine(ys_bf, pos0, pos1):
    return _build_sc_kernels()["combine"](ys_bf, pos0, pos1)


def _gelu_exact(h):
    return 0.5 * h * (1.0 + lax.erf(h * _INV_SQRT2))


def _mlp_body(pe_ref, xs_ref, rw_ref, w1_ref, b1_ref, w2_ref, b2_ref, ys_ref):
    b = pl.program_id(0)

    @pl.when(b < pe_ref[NB])
    def _():
        h = lax.dot_general(
            xs_ref[...], w1_ref[0], (((1,), (1,)), ((), ())),
            preferred_element_type=jnp.float32) + b1_ref[0]
        g = _gelu_exact(h).astype(jnp.bfloat16)
        y = lax.dot_general(
            g, w2_ref[0], (((1,), (1,)), ((), ())),
            preferred_element_type=jnp.float32)
        ys_ref[...] = (y + b2_ref[0]) * rw_ref[...]


def _mlp(pe_plus, xs_bf, rw, w1b, b1r, w2b, b2r):
    grid_spec = pltpu.PrefetchScalarGridSpec(
        num_scalar_prefetch=1,
        grid=(NB,),
        in_specs=[
            pl.BlockSpec((BM, DIM), lambda b, pe: (b, 0)),
            pl.BlockSpec((BM, 1), lambda b, pe: (b, 0)),
            pl.BlockSpec((1, HID, DIM), lambda b, pe: (pe[b], 0, 0)),
            pl.BlockSpec((1, 1, HID), lambda b, pe: (pe[b], 0, 0)),
            pl.BlockSpec((1, DIM, HID), lambda b, pe: (pe[b], 0, 0)),
            pl.BlockSpec((1, 1, DIM), lambda b, pe: (pe[b], 0, 0)),
        ],
        out_specs=pl.BlockSpec((BM, DIM), lambda b, pe: (b, 0)),
    )
    return pl.pallas_call(
        _mlp_body,
        grid_spec=grid_spec,
        out_shape=jax.ShapeDtypeStruct((NPAD, DIM), jnp.float32),
    )(pe_plus, xs_bf, rw, w1b, b1r, w2b, b2r)


def _pack_i32(a_bf16):
    n, d = a_bf16.shape
    return lax.bitcast_convert_type(
        a_bf16.reshape(n, d // 2, 2), jnp.int32)


def _unpack_bf16(a_i32):
    n, w = a_i32.shape
    return lax.bitcast_convert_type(a_i32, jnp.bfloat16).reshape(n, 2 * w)


@jax.jit
def _moe_dispatch(x, gate_w, gate_b2d, w1b, b1r, w2b, b2r):
    ev, wv = _gating(x, gate_w, gate_b2d)
    pe_plus, row_token, rw, pos0, pos1 = _route_meta(ev, wv)
    x_i32 = _pack_i32(x.astype(jnp.bfloat16))
    xs_i32 = _sc_gather(x_i32, row_token)
    ys = _mlp(pe_plus, _unpack_bf16(xs_i32), rw, w1b, b1r, w2b, b2r)
    return _sc_combine(ys, pos0, pos1)


def kernel(x, gate_w, gate_b, w1, b1, w2, b2):
    w1b = w1.astype(jnp.bfloat16)
    w2b = w2.astype(jnp.bfloat16)
    return _moe_dispatch(x, gate_w, gate_b.reshape(1, E), w1b,
                         b1.reshape(E, 1, HID), w2b, b2.reshape(E, 1, DIM))


# speedup vs baseline: 1.3654x; 1.3654x over previous
import functools

import jax
import jax.numpy as jnp
from jax import lax
from jax.experimental import pallas as pl
from jax.experimental.pallas import tpu as pltpu
from jax.experimental.pallas import tpu_sc as plsc

T = 2048
DIM = 1024
HID = 4096
E = 8
TOP_K = 2

BM = 256
NPAD = 6144
NB = NPAD // BM

NC = 2
NS = 16
NW = NC * NS

_C_PER_W = T // NW

_INV_SQRT2 = 0.7071067811865476


def _gate_body(x_ref, gw_ref, gb_ref, ev_ref, wv_ref):
    xf = x_ref[...]
    logits = lax.dot_general(
        gw_ref[...], xf, (((1,), (1,)), ((), ())),
        preferred_element_type=jnp.float32) + gb_ref[...]
    mx = jnp.max(logits, axis=0, keepdims=True)
    ex = jnp.exp(logits - mx)
    p = ex / jnp.sum(ex, axis=0, keepdims=True)
    iota8 = lax.broadcasted_iota(jnp.int32, p.shape, 0)
    v1 = jnp.max(p, axis=0, keepdims=True)
    i1 = jnp.min(jnp.where(p >= v1, iota8, E), axis=0, keepdims=True)
    m1 = iota8 == i1
    p2 = jnp.where(m1, -jnp.inf, p)
    v2 = jnp.max(p2, axis=0, keepdims=True)
    i2 = jnp.min(jnp.where(p2 >= v2, iota8, E), axis=0, keepdims=True)
    ev_ref[...] = jnp.concatenate([i1, i2], axis=0)
    wv_ref[...] = jnp.concatenate([v1, v2], axis=0)


def _gating(x, gate_w, gate_b2d):
    return pl.pallas_call(
        _gate_body,
        grid=(1,),
        in_specs=[
            pl.BlockSpec((T, DIM), lambda i: (0, 0)),
            pl.BlockSpec((E, DIM), lambda i: (0, 0)),
            pl.BlockSpec((E, 1), lambda i: (0, 0)),
        ],
        out_specs=[
            pl.BlockSpec((TOP_K, T), lambda i: (0, 0)),
            pl.BlockSpec((TOP_K, T), lambda i: (0, 0)),
        ],
        out_shape=[
            jax.ShapeDtypeStruct((TOP_K, T), jnp.int32),
            jax.ShapeDtypeStruct((TOP_K, T), jnp.float32),
        ],
    )(x, gate_w, gate_b2d)


def _route_meta(ev):
    fe = ev.reshape(-1)
    oh = fe[:, None] == jnp.arange(E, dtype=jnp.int32)[None, :]
    csum = jnp.cumsum(oh.astype(jnp.int32), axis=0)
    counts = csum[-1]
    rank = jnp.sum(jnp.where(oh, csum - 1, 0), axis=1)
    padded = ((counts + BM - 1) // BM) * BM
    ends = jnp.cumsum(padded)
    offs = ends - padded
    dest = (offs[fe] + rank).astype(jnp.int32)
    dest2 = dest.reshape(TOP_K, T)
    tok = jnp.arange(2 * T, dtype=jnp.int32) % T
    row_token = jnp.zeros((NPAD,), jnp.int32).at[dest].set(tok)
    bstart = jnp.arange(NB, dtype=jnp.int32) * BM
    block_expert = jnp.minimum(
        jnp.sum(bstart[:, None] >= ends[None, :], axis=1), E - 1
    ).astype(jnp.int32)
    nactive = (ends[-1] // BM).astype(jnp.int32)
    pe_plus = jnp.concatenate([block_expert, nactive[None]])
    return pe_plus, dest2, row_token


_SC_BUILT = {}


def _build_sc_kernels():
    if "dispatch" in _SC_BUILT:
        return _SC_BUILT
    mesh = plsc.VectorSubcoreMesh(core_axis_name="c", subcore_axis_name="s")

    GCH = 96
    _G_PER_W = NPAD // NW

    @functools.partial(
        pl.kernel, mesh=mesh,
        out_type=jax.ShapeDtypeStruct((NPAD, DIM), jnp.float32),
        scratch_types=[
            pltpu.VMEM((GCH,), jnp.int32),
            pltpu.VMEM((GCH, DIM), jnp.float32),
            pltpu.SemaphoreType.DMA,
        ],
    )
    def sc_dispatch(x_hbm, rowtok_hbm, xs_hbm, idx_v, rows, sem):
        wid = lax.axis_index("s") * NC + lax.axis_index("c")
        base = wid * _G_PER_W
        for ci in range(_G_PER_W // GCH):
            cbase = base + ci * GCH
            pltpu.sync_copy(rowtok_hbm.at[pl.ds(cbase, GCH)], idx_v)
            pltpu.async_copy(x_hbm.at[idx_v], rows, sem).wait()
            pltpu.sync_copy(rows, xs_hbm.at[pl.ds(cbase, GCH)])

    CHK = 32

    @functools.partial(
        pl.kernel, mesh=mesh,
        out_type=jax.ShapeDtypeStruct((T, DIM), jnp.float32),
        scratch_types=[
            pltpu.VMEM((CHK,), jnp.int32),
            pltpu.VMEM((CHK,), jnp.int32),
            pltpu.VMEM((CHK, 16), jnp.float32),
            pltpu.VMEM((CHK, 16), jnp.float32),
            pltpu.VMEM((CHK, DIM), jnp.float32),
            pltpu.VMEM((CHK, DIM), jnp.float32),
            pltpu.SemaphoreType.DMA,
        ],
    )
    def sc_combine(ys_hbm, dest2_hbm, wv16_hbm, out_hbm,
                   posa, posb, wa, wb, rows_a, rows_b, sem):
        wid = lax.axis_index("s") * NC + lax.axis_index("c")
        tokbase = wid * _C_PER_W
        for ci in range(_C_PER_W // CHK):
            cbase = tokbase + ci * CHK
            pltpu.sync_copy(dest2_hbm.at[0, pl.ds(cbase, CHK)], posa)
            pltpu.sync_copy(dest2_hbm.at[1, pl.ds(cbase, CHK)], posb)
            pltpu.sync_copy(wv16_hbm.at[0, pl.ds(cbase, CHK)], wa)
            pltpu.sync_copy(wv16_hbm.at[1, pl.ds(cbase, CHK)], wb)
            cpa = pltpu.async_copy(ys_hbm.at[posa], rows_a, sem)
            cpb = pltpu.async_copy(ys_hbm.at[posb], rows_b, sem)
            cpa.wait()
            cpb.wait()

            def addrow(r, carry):
                w0 = wa[r, :]
                w1 = wb[r, :]
                for u in range(DIM // 16):
                    sl = pl.ds(u * 16, 16)
                    rows_a[r, sl] = rows_a[r, sl] * w0 + rows_b[r, sl] * w1
                return carry

            lax.fori_loop(0, CHK, addrow, 0)
            pltpu.sync_copy(rows_a, out_hbm.at[pl.ds(cbase, CHK)])

    _SC_BUILT["dispatch"] = sc_dispatch
    _SC_BUILT["combine"] = sc_combine
    return _SC_BUILT


def _sc_dispatch(x, row_token):
    return _build_sc_kernels()["dispatch"](x, row_token)


def _sc_combine(ys, dest2, wv16):
    return _build_sc_kernels()["combine"](ys, dest2, wv16)


def _gelu_exact(h):
    return 0.5 * h * (1.0 + lax.erf(h * _INV_SQRT2))


def _mlp_body(pe_ref, xs_ref, w1_ref, b1_ref, w2_ref, b2_ref, ys_ref):
    b = pl.program_id(0)

    @pl.when(b < pe_ref[NB])
    def _():
        h = lax.dot_general(
            xs_ref[...].astype(jnp.bfloat16), w1_ref[0],
            (((1,), (1,)), ((), ())),
            preferred_element_type=jnp.float32) + b1_ref[0]
        g = _gelu_exact(h).astype(jnp.bfloat16)
        y = lax.dot_general(
            g, w2_ref[0], (((1,), (1,)), ((), ())),
            preferred_element_type=jnp.float32)
        ys_ref[...] = y + b2_ref[0]


def _mlp(pe_plus, xs, w1b, b1r, w2b, b2r):
    grid_spec = pltpu.PrefetchScalarGridSpec(
        num_scalar_prefetch=1,
        grid=(NB,),
        in_specs=[
            pl.BlockSpec((BM, DIM), lambda b, pe: (b, 0)),
            pl.BlockSpec((1, HID, DIM), lambda b, pe: (pe[b], 0, 0)),
            pl.BlockSpec((1, 1, HID), lambda b, pe: (pe[b], 0, 0)),
            pl.BlockSpec((1, DIM, HID), lambda b, pe: (pe[b], 0, 0)),
            pl.BlockSpec((1, 1, DIM), lambda b, pe: (pe[b], 0, 0)),
        ],
        out_specs=pl.BlockSpec((BM, DIM), lambda b, pe: (b, 0)),
    )
    return pl.pallas_call(
        _mlp_body,
        grid_spec=grid_spec,
        out_shape=jax.ShapeDtypeStruct((NPAD, DIM), jnp.float32),
    )(pe_plus, xs, w1b, b1r, w2b, b2r)


@jax.jit
def _moe_dispatch(x, gate_w, gate_b2d, w1b, b1r, w2b, b2r):
    ev, wv2 = _gating(x, gate_w, gate_b2d)
    pe_plus, dest2, row_token = _route_meta(ev)
    wv16 = jnp.broadcast_to(wv2[:, :, None], (TOP_K, T, 16))
    xs = _sc_dispatch(x, row_token)
    ys = _mlp(pe_plus, xs, w1b, b1r, w2b, b2r)
    return _sc_combine(ys, dest2, wv16)


def kernel(x, gate_w, gate_b, w1, b1, w2, b2):
    w1b = w1.astype(jnp.bfloat16)
    w2b = w2.astype(jnp.bfloat16)
    return _moe_dispatch(x, gate_w, gate_b.reshape(E, 1), w1b,
                         b1.reshape(E, 1, HID), w2b, b2.reshape(E, 1, DIM))
